# speedup vs baseline: 3.1503x; 3.1503x over previous
import jax
import jax.numpy as jnp
import numpy as np
from jax.experimental import pallas as pl

RADIUS = 0.1
DILATION = 0.05
NDIM = 3
KS = 3
IN_CH = 64
OUT_CH = 64
TI = 256


def _cell_offsets():
    g = (np.arange(KS) - (KS - 1) / 2.0) * DILATION
    mesh = np.stack(np.meshgrid(*([g] * NDIM), indexing="ij"), axis=-1)
    return mesh.reshape(-1, NDIM)


_OFFS = _cell_offsets()


def _conv_kernel(locs_tile_ref, locs_t_ref, data_ref, den_ref, wkt_ref, b_ref,
                 out_ref):
    li = locs_tile_ref[:]
    lx, ly, lz = li[:, 0:1], li[:, 1:2], li[:, 2:3]
    jx = locs_t_ref[0:1, :]
    jy = locs_t_ref[1:2, :]
    jz = locs_t_ref[2:3, :]
    dx = lx - jx
    dy = ly - jy
    dz = lz - jz
    d2 = dx * dx + dy * dy + dz * dz

    dscaled = data_ref[:] * (1.0 / den_ref[:])

    inv_h = 1.0 / RADIUS
    acc = jnp.zeros((TI, OUT_CH), dtype=jnp.float32)
    for k in range(_OFFS.shape[0]):
        ox, oy, oz = (float(v) for v in _OFFS[k])
        r2 = d2
        if ox != 0.0:
            r2 = r2 + (2.0 * ox) * dx
        if oy != 0.0:
            r2 = r2 + (2.0 * oy) * dy
        if oz != 0.0:
            r2 = r2 + (2.0 * oz) * dz
        c = ox * ox + oy * oy + oz * oz
        r = jnp.sqrt(r2 + (c + 1e-12))
        u = jnp.maximum(1.0 - r * inv_h, 0.0)
        w = u * u * u
        f = jnp.dot(w, dscaled, preferred_element_type=jnp.float32)
        acc = acc + jnp.dot(f, wkt_ref[k], preferred_element_type=jnp.float32)

    out_ref[:] = acc + b_ref[:]


@jax.jit
def kernel(locs, data, density, W, b):
    B, n, _ = locs.shape
    locs2 = locs.reshape(n, NDIM)
    locs_t = locs2.T
    data2 = data.reshape(n, IN_CH)
    den2 = density.reshape(n, 1)
    wkt = jnp.transpose(W, (2, 1, 0))
    b2 = b.reshape(1, OUT_CH)

    grid = (n // TI,)
    out = pl.pallas_call(
        _conv_kernel,
        grid=grid,
        in_specs=[
            pl.BlockSpec((TI, NDIM), lambda i: (i, 0)),
            pl.BlockSpec((NDIM, n), lambda i: (0, 0)),
            pl.BlockSpec((n, IN_CH), lambda i: (0, 0)),
            pl.BlockSpec((n, 1), lambda i: (0, 0)),
            pl.BlockSpec((_OFFS.shape[0], IN_CH, OUT_CH), lambda i: (0, 0, 0)),
            pl.BlockSpec((1, OUT_CH), lambda i: (0, 0)),
        ],
        out_specs=pl.BlockSpec((TI, OUT_CH), lambda i: (i, 0)),
        out_shape=jax.ShapeDtypeStruct((n, OUT_CH), jnp.float32),
    )(locs2, locs_t, data2, den2, wkt, b2)
    return out.reshape(B, n, OUT_CH)
